# Initial kernel scaffold; baseline (speedup 1.0000x reference)
import jax
import jax.numpy as jnp
from jax.experimental import pallas as pl


def kernel(x, edge_index, W, att, bias):
    raise NotImplementedError("write your pallas kernel here")


# speedup vs baseline: 53.8144x; 53.8144x over previous
import functools

import jax
import jax.numpy as jnp
import numpy as np
from jax import lax
from jax.experimental import pallas as pl
from jax.experimental.pallas import tpu as pltpu
from jax.experimental.pallas import tpu_sc as plsc

N_NODES = 10000
IN = 128
H = 8
C = 16
HC = H * C
NEG = 0.2

NC = 2
NS = 16
NW = NC * NS
NPAD = 10240
NPAD8 = NPAD // 8
ZROWS = NPAD // NS
DROWS = NPAD8 // NS
CHUNK = 40

BN = 400
NBLK = N_NODES // BN
SKIP_SC = False
LEVEL = 7
DIAG = 0

_KRON = np.kron(np.eye(H, dtype=np.float32), np.ones((1, C), np.float32))
_R16 = np.zeros((16, HC), np.float32)
_R16[:H] = _KRON
_R128 = np.zeros((HC, HC), np.float32)
_R128[:H] = _KRON


def _prep_body(x_ref, w_ref, asrc_ref, adst_ref,
               xt_ref, ssrc_ref, sdst_ref, m_ref, mscr):
    i = pl.program_id(0)
    xt = jnp.dot(x_ref[...], w_ref[...], preferred_element_type=jnp.float32)
    ssrc = jnp.dot(xt, asrc_ref[...], preferred_element_type=jnp.float32)
    sdst = jnp.dot(xt, adst_ref[...], preferred_element_type=jnp.float32)
    xt_ref[...] = xt
    ssrc_ref[...] = ssrc
    sdst_ref[...] = sdst
    bs = jnp.max(ssrc)
    bd = jnp.max(sdst)

    @pl.when(i == 0)
    def _():
        mscr[0] = bs
        mscr[1] = bd

    @pl.when(i > 0)
    def _():
        mscr[0] = jnp.maximum(mscr[0], bs)
        mscr[1] = jnp.maximum(mscr[1], bd)

    @pl.when(i == pl.num_programs(0) - 1)
    def _():
        m = mscr[0] + mscr[1]
        m = jnp.where(m > 0, m, NEG * m)
        m_ref[...] = jnp.full((1, HC), m, jnp.float32)


_prep_call = pl.pallas_call(
    _prep_body,
    grid=(NBLK,),
    in_specs=[
        pl.BlockSpec((BN, IN), lambda i: (i, 0)),
        pl.BlockSpec((IN, HC), lambda i: (0, 0)),
        pl.BlockSpec((IN, HC), lambda i: (0, 0)),
        pl.BlockSpec((IN, HC), lambda i: (0, 0)),
    ],
    out_specs=[
        pl.BlockSpec((BN, HC), lambda i: (i, 0)),
        pl.BlockSpec((BN, HC), lambda i: (i, 0)),
        pl.BlockSpec((BN, HC), lambda i: (i, 0)),
        pl.BlockSpec((1, HC), lambda i: (0, 0)),
    ],
    out_shape=[
        jax.ShapeDtypeStruct((N_NODES, HC), jnp.float32),
        jax.ShapeDtypeStruct((N_NODES, HC), jnp.float32),
        jax.ShapeDtypeStruct((N_NODES, HC), jnp.float32),
        jax.ShapeDtypeStruct((1, HC), jnp.float32),
    ],
    scratch_shapes=[pltpu.SMEM((2,), jnp.float32)],
)


def _make_edge_call(n_edges):
    epw = n_edges // NW
    nchunks = epw // CHUNK
    mesh = plsc.VectorSubcoreMesh(
        core_axis_name="c", subcore_axis_name="s",
        num_cores=NC, num_subcores=NS)

    @functools.partial(
        pl.kernel,
        out_type=[
            jax.ShapeDtypeStruct((NC, NPAD, HC), jnp.float32),
            jax.ShapeDtypeStruct((NC, NPAD8, HC), jnp.float32),
        ],
        mesh=mesh,
        scratch_types=[
            pltpu.VMEM((CHUNK,), jnp.int32),
            pltpu.VMEM((CHUNK,), jnp.int32),
            pltpu.VMEM((CHUNK, HC), jnp.float32),
            pltpu.VMEM((CHUNK, HC), jnp.float32),
            pltpu.VMEM((CHUNK, HC), jnp.float32),
            pltpu.VMEM((CHUNK, HC), jnp.float32),
            pltpu.VMEM((16,), jnp.float32),
            pltpu.VMEM((CHUNK,), jnp.int32),
            pltpu.VMEM((CHUNK,), jnp.int32),
            pltpu.VMEM_SHARED((NPAD, HC), jnp.float32),
            pltpu.VMEM_SHARED((NPAD8, HC), jnp.float32),
            pltpu.SemaphoreType.DMA,
            pltpu.SemaphoreType.DMA,
            pltpu.SemaphoreType.DMA,
        ],
    )
    def edge_kernel(ei_hbm, xt_hbm, ssrc_hbm, sdst_hbm, m_hbm, zn_hbm, zd_hbm,
                    num_out, den_out,
                    srcv, dstv, ssrcv, sdstv, xtv, pv, mv, rowv, drowv,
                    num_sh, den_sh, sem0, sem1, sem2):
        cid = lax.axis_index("c")
        sid = lax.axis_index("s")
        wid = sid * NC + cid

        zvec = jnp.zeros((16,), jnp.float32)

        def zrow(j, _):
            for k in range(HC // 16):
                xtv[j, pl.ds(k * 16, 16)] = zvec
            return 0

        lax.fori_loop(0, CHUNK, zrow, 0)
        lane = lax.iota(jnp.int32, 16)

        def set_rows(base):
            for off in (0, 16, CHUNK - 16):
                rowv[pl.ds(off, 16)] = base + off + lane

        def zcopy(t, _):
            base = sid * ZROWS + t * CHUNK
            set_rows(base)
            pltpu.sync_copy(xtv, num_sh.at[rowv])
            return 0

        def zcopy_d(t, _):
            base = sid * DROWS + t * CHUNK
            set_rows(base)
            pltpu.sync_copy(xtv, den_sh.at[rowv])
            return 0

        if LEVEL >= 2:
            lax.fori_loop(0, ZROWS // CHUNK, zcopy, 0)
            lax.fori_loop(0, DROWS // CHUNK, zcopy_d, 0)
        if LEVEL >= 3:
            plsc.subcore_barrier()

        if LEVEL >= 4:
            pltpu.sync_copy(m_hbm.at[0, pl.ds(0, 16)], mv)
        mvec = mv[...]

        ebase = wid * epw

        def chunk_body(t, _):
            b = ebase + t * CHUNK
            pltpu.sync_copy(ei_hbm.at[pl.ds(b, CHUNK)], srcv)
            pltpu.sync_copy(ei_hbm.at[pl.ds(n_edges + b, CHUNK)], dstv)
            cp1 = pltpu.async_copy(ssrc_hbm.at[srcv], ssrcv, sem0)
            cp2 = pltpu.async_copy(sdst_hbm.at[dstv], sdstv, sem1)
            cp3 = pltpu.async_copy(xt_hbm.at[srcv], xtv, sem2)
            cp1.wait()
            cp2.wait()
            cp3.wait()

            if LEVEL >= 6:
                for off in (0, 16, CHUNK - 16):
                    drowv[pl.ds(off, 16)] = lax.shift_right_logical(
                        dstv[pl.ds(off, 16)], 3)
                for e in range(CHUNK):
                    a = ssrcv[e, pl.ds(0, 16)] + sdstv[e, pl.ds(0, 16)]
                    a = jnp.where(a > 0, a, NEG * a)
                    p = jnp.exp(a - mvec)
                    pc = jnp.where(lane < H, p, zvec)
                    woff = min(e - (e % 16), CHUNK - 16)
                    sub = jnp.bitwise_and(dstv[pl.ds(woff, 16)][e - woff], 7)
                    for k in range(HC // 16):
                        pv[e, pl.ds(k * 16, 16)] = jnp.where(sub == k, pc, zvec)
                    for h in range(H):
                        xtv[e, pl.ds(h * C, C)] = xtv[e, pl.ds(h * C, C)] * p[h]
            if LEVEL >= 7:
                pltpu.sync_copy(xtv, num_sh.at[dstv], add=True)
                pltpu.sync_copy(pv, den_sh.at[drowv], add=True)
            return 0

        if LEVEL >= 5:
            lax.fori_loop(0, nchunks, chunk_body, 0)
        if LEVEL >= 3:
            plsc.subcore_barrier()

        def wb(t, _):
            base = sid * ZROWS + t * CHUNK
            if LEVEL >= 2:
                set_rows(base)
                pltpu.async_copy(num_sh.at[rowv], xtv, sem0).wait()
            pltpu.sync_copy(xtv, num_out.at[cid, pl.ds(base, CHUNK)])
            return 0

        def wb_d(t, _):
            base = sid * DROWS + t * CHUNK
            if LEVEL >= 2:
                set_rows(base)
                pltpu.async_copy(den_sh.at[rowv], xtv, sem0).wait()
            pltpu.sync_copy(xtv, den_out.at[cid, pl.ds(base, CHUNK)])
            return 0

        lax.fori_loop(0, ZROWS // CHUNK, wb, 0)
        lax.fori_loop(0, DROWS // CHUNK, wb_d, 0)

    return edge_kernel


def _combine_body(num_ref, den_ref, ssrc_ref, sdst_ref, xt_ref,
                  m_ref, r8_ref, r128_ref, b_ref, out_ref):
    s = ssrc_ref[...] + sdst_ref[...]
    a = jnp.where(s > 0, s, NEG * s)
    p128 = jnp.exp(a - m_ref[...])
    pw = jnp.dot(p128, r128_ref[...], preferred_element_type=jnp.float32)
    den16 = den_ref[0] + den_ref[1]
    denw = jnp.dot(den16, r8_ref[...], preferred_element_type=jnp.float32) + pw
    num = num_ref[0] + num_ref[1] + pw * xt_ref[...]
    out_ref[...] = num / denw + b_ref[...]
    if DIAG == 1:
        out_ref[...] = num_ref[0] + num_ref[1]
    elif DIAG == 2:
        out_ref[...] = jnp.dot(den_ref[0] + den_ref[1], r8_ref[...],
                               preferred_element_type=jnp.float32)


_combine_call = pl.pallas_call(
    _combine_body,
    grid=(NBLK,),
    in_specs=[
        pl.BlockSpec((NC, BN, HC), lambda i: (0, i, 0)),
        pl.BlockSpec((NC, BN, 16), lambda i: (0, i, 0)),
        pl.BlockSpec((BN, HC), lambda i: (i, 0)),
        pl.BlockSpec((BN, HC), lambda i: (i, 0)),
        pl.BlockSpec((BN, HC), lambda i: (i, 0)),
        pl.BlockSpec((1, HC), lambda i: (0, 0)),
        pl.BlockSpec((16, HC), lambda i: (0, 0)),
        pl.BlockSpec((HC, HC), lambda i: (0, 0)),
        pl.BlockSpec((1, HC), lambda i: (0, 0)),
    ],
    out_specs=pl.BlockSpec((BN, HC), lambda i: (i, 0)),
    out_shape=jax.ShapeDtypeStruct((N_NODES, HC), jnp.float32),
)


def kernel(x, edge_index, W, att, bias):
    x = x.astype(jnp.float32)
    ei = edge_index.astype(jnp.int32)

    a_src = att[0, :, :C]
    a_dst = att[0, :, C:]
    eye8 = jnp.eye(H, dtype=jnp.float32)
    A_src = jnp.einsum('hg,gc->hcg', eye8, a_src).reshape(IN, H)
    A_dst = jnp.einsum('hg,gc->hcg', eye8, a_dst).reshape(IN, H)
    zpad = jnp.zeros((IN, HC - H), jnp.float32)
    A_src = jnp.concatenate([A_src, zpad], axis=1)
    A_dst = jnp.concatenate([A_dst, zpad], axis=1)

    xt, ssrc, sdst, m = _prep_call(x, W, A_src, A_dst)

    edge_call = _make_edge_call(ei.shape[1])
    zn = jnp.zeros((ZROWS, HC), jnp.float32)
    zd = jnp.zeros((ZROWS, 16), jnp.float32)
    if SKIP_SC:
        num_p = jnp.zeros((NC, NPAD, HC), jnp.float32)
        den_p = jnp.zeros((NC, NPAD, 16), jnp.float32)
    else:
        num_p, den_p = edge_call(ei.reshape(-1), xt, ssrc, sdst, m, zn, zd)
        den_p = den_p.reshape(NC, NPAD, 16)

    out = _combine_call(num_p, den_p, ssrc, sdst, xt, m,
                        jnp.asarray(_R16), jnp.asarray(_R128),
                        bias.reshape(1, HC).astype(jnp.float32))
    return out
